# Initial kernel scaffold; baseline (speedup 1.0000x reference)
import jax
import jax.numpy as jnp
from jax import lax
from jax.experimental import pallas as pl
from jax.experimental.pallas import tpu as pltpu
from jax.experimental.pallas import tpu_sc as plsc

VALUES_PER_CHANNEL = 256
C = 3
D = 256
L = 16
K = 128


def _build(M):
    NW = 32
    b_per_w = M // NW
    G = b_per_w // K
    assert b_per_w % K == 0 and M % NW == 0

    mesh = plsc.VectorSubcoreMesh(core_axis_name="c", subcore_axis_name="s")

    @pl.kernel(
        out_type=jax.ShapeDtypeStruct((M, D), jnp.float32),
        mesh=mesh,
        scratch_types=[
            pltpu.VMEM((G, K), jnp.int32),
            pltpu.VMEM((K, D), jnp.float32),
            pltpu.VMEM((K, D), jnp.float32),
            pltpu.SemaphoreType.DMA,
            pltpu.SemaphoreType.DMA,
        ],
    )
    def k(idx_hbm, table_hbm, out_hbm, idx_v, rows0, rows1, sem0, sem1):
        cid = lax.axis_index("c")
        sid = lax.axis_index("s")
        wid = sid * 2 + cid
        row0 = wid * G

        pltpu.sync_copy(idx_hbm.at[pl.ds(row0, G), :], idx_v)

        iota = lax.iota(jnp.int32, L)

        def adjust(g):
            for j in range(K // L):
                base_m = (row0 + g) * K + j * L
                off = ((base_m + iota) % C) * VALUES_PER_CHANNEL
                idx_v[g, pl.ds(j * L, L)] = idx_v[g, pl.ds(j * L, L)] + off

        def start_gather(g, buf, sem):
            pltpu.async_copy(table_hbm.at[idx_v.at[g]], buf, sem)

        def wait_gather(g, buf, sem):
            pltpu.make_async_copy(table_hbm.at[idx_v.at[g]], buf, sem).wait()

        def write_out(g, buf):
            pltpu.sync_copy(buf, out_hbm.at[pl.ds((row0 + g) * K, K), :])

        adjust(0)
        start_gather(0, rows0, sem0)

        @pl.loop(0, G, step=2)
        def _(go):
            adjust(go + 1)
            start_gather(go + 1, rows1, sem1)
            wait_gather(go, rows0, sem0)
            write_out(go, rows0)

            @pl.when(go + 2 < G)
            def _():
                adjust(go + 2)
                start_gather(go + 2, rows0, sem0)

            wait_gather(go + 1, rows1, sem1)
            write_out(go + 1, rows1)

    return k


def kernel(inputs, table):
    shp = inputs.shape
    M = inputs.size
    idx2d = inputs.reshape(M // K, K)
    out = _build(M)(idx2d, table)
    return out.reshape(shp[:-1] + (shp[-1] * D,))


# speedup vs baseline: 3.3593x; 3.3593x over previous
import jax
import jax.numpy as jnp
from jax import lax
from jax.experimental import pallas as pl
from jax.experimental.pallas import tpu as pltpu
from jax.experimental.pallas import tpu_sc as plsc

VALUES_PER_CHANNEL = 256
C = 3
D = 256
L = 16
K = 128


def _build(M):
    NW = 32
    b_per_w = M // NW
    G = b_per_w // K
    assert b_per_w % K == 0 and M % NW == 0

    mesh = plsc.VectorSubcoreMesh(core_axis_name="c", subcore_axis_name="s")

    @pl.kernel(
        out_type=jax.ShapeDtypeStruct((M, D), jnp.float32),
        mesh=mesh,
        scratch_types=[
            pltpu.VMEM((G, K), jnp.int32),
            pltpu.VMEM((K, D), jnp.float32),
            pltpu.VMEM((K, D), jnp.float32),
            pltpu.SemaphoreType.DMA,
            pltpu.SemaphoreType.DMA,
        ],
    )
    def k(idx_hbm, table_hbm, out_hbm, idx_v, rows0, rows1, sem0, sem1):
        cid = lax.axis_index("c")
        sid = lax.axis_index("s")
        wid = sid * 2 + cid
        row0 = wid * G

        pltpu.sync_copy(idx_hbm.at[wid], idx_v)

        iota = lax.iota(jnp.int32, L)

        def adjust(g):
            for j in range(K // L):
                base_m = (row0 + g) * K + j * L
                off = ((base_m + iota) % C) * VALUES_PER_CHANNEL
                idx_v[g, pl.ds(j * L, L)] = idx_v[g, pl.ds(j * L, L)] + off

        def start_gather(g, buf, sem):
            pltpu.async_copy(table_hbm.at[idx_v.at[g]], buf, sem)

        def wait_gather(g, buf, sem):
            pltpu.make_async_copy(table_hbm.at[idx_v.at[g]], buf, sem).wait()

        def write_out(g, buf):
            pltpu.sync_copy(buf, out_hbm.at[pl.ds((row0 + g) * K, K), :])

        adjust(0)
        start_gather(0, rows0, sem0)

        @pl.loop(0, G, step=2)
        def _(go):
            adjust(go + 1)
            start_gather(go + 1, rows1, sem1)
            wait_gather(go, rows0, sem0)
            write_out(go, rows0)

            @pl.when(go + 2 < G)
            def _():
                adjust(go + 2)
                start_gather(go + 2, rows0, sem0)

            wait_gather(go + 1, rows1, sem1)
            write_out(go + 1, rows1)

    return k


def kernel(inputs, table):
    shp = inputs.shape
    M = inputs.size
    idx3d = inputs.reshape(32, M // (32 * K), K)
    out = _build(M)(idx3d, table)
    return out.reshape(shp[:-1] + (shp[-1] * D,))


# speedup vs baseline: 3.6642x; 1.0908x over previous
import jax
import jax.numpy as jnp
from jax import lax
from jax.experimental import pallas as pl
from jax.experimental.pallas import tpu as pltpu
from jax.experimental.pallas import tpu_sc as plsc

VALUES_PER_CHANNEL = 256
C = 3
D = 256
L = 16
K = 48
NB = 4


def _build(M):
    NW = 32
    b_per_w = M // NW
    G = b_per_w // K
    assert b_per_w % K == 0 and M % NW == 0 and G % NB == 0

    mesh = plsc.VectorSubcoreMesh(core_axis_name="c", subcore_axis_name="s")

    @pl.kernel(
        out_type=jax.ShapeDtypeStruct((M, D), jnp.float32),
        mesh=mesh,
        compiler_params=pltpu.CompilerParams(use_tc_tiling_on_sc=False),
        scratch_types=[
            pltpu.VMEM((G, K), jnp.int32),
            [pltpu.VMEM((K, D), jnp.float32) for _ in range(NB)],
            [pltpu.SemaphoreType.DMA for _ in range(NB)],
            [pltpu.SemaphoreType.DMA for _ in range(NB)],
        ],
    )
    def k(idx_hbm, table_hbm, out_hbm, idx_v, bufs, gsems, wsems):
        cid = lax.axis_index("c")
        sid = lax.axis_index("s")
        wid = sid * 2 + cid
        row0 = wid * G

        pltpu.sync_copy(idx_hbm.at[wid], idx_v)

        iota = lax.iota(jnp.int32, L)

        def adjust(g):
            for j in range(K // L):
                base_m = (row0 + g) * K + j * L
                off = ((base_m + iota) % C) * VALUES_PER_CHANNEL
                idx_v[g, pl.ds(j * L, L)] = idx_v[g, pl.ds(j * L, L)] + off

        def gather(g, b):
            return pltpu.make_async_copy(table_hbm.at[idx_v.at[g]], bufs[b],
                                         gsems[b])

        def write(g, b):
            return pltpu.make_async_copy(
                bufs[b], out_hbm.at[pl.ds((row0 + g) * K, K), :], wsems[b])

        adjust(0)
        gather(0, 0).start()
        adjust(1)
        gather(1, 1).start()

        @pl.loop(0, G, step=NB)
        def _(go):
            for b in range(NB):
                g = go + b
                bn = (b + 2) % NB

                @pl.when(g + 2 < G)
                def _():
                    @pl.when(g >= 2)
                    def _():
                        write(g - 2, bn).wait()
                    adjust(g + 2)
                    gather(g + 2, bn).start()

                gather(g, b).wait()
                write(g, b).start()

        for t in range(NB):
            g = G - NB + t
            write(g, g % NB).wait()

    return k


def kernel(inputs, table):
    shp = inputs.shape
    M = inputs.size
    idx3d = inputs.reshape(32, M // (32 * K), K)
    out = _build(M)(idx3d, table)
    return out.reshape(shp[:-1] + (shp[-1] * D,))


# speedup vs baseline: 7.3471x; 2.0051x over previous
import jax
import jax.numpy as jnp
from jax import lax
from jax.experimental import pallas as pl
from jax.experimental.pallas import tpu as pltpu
from jax.experimental.pallas import tpu_sc as plsc

VALUES_PER_CHANNEL = 256
C = 3
D = 256
L = 16
K = C * L
NB = 4
NW = 32
R = 1792
W = 224
KB = W // L
NS = 14
SC_ = 56


def _build():
    rows_per_w = R // NW
    G = rows_per_w * KB
    assert G == NS * SC_ and SC_ % NB == 0

    mesh = plsc.VectorSubcoreMesh(core_axis_name="c", subcore_axis_name="s")

    @pl.kernel(
        out_type=jax.ShapeDtypeStruct((R, W, C * D), jnp.float32),
        mesh=mesh,
        scratch_types=[
            [pltpu.VMEM((SC_, K), jnp.int32) for _ in range(2)],
            [pltpu.VMEM((K, D), jnp.float32) for _ in range(NB)],
            [pltpu.SemaphoreType.DMA for _ in range(2)],
            [pltpu.SemaphoreType.DMA for _ in range(NB)],
            [pltpu.SemaphoreType.DMA for _ in range(NB)],
        ],
    )
    def k(idx_hbm, table_hbm, out_hbm, ibufs, bufs, isems, gsems, wsems):
        cid = lax.axis_index("c")
        sid = lax.axis_index("s")
        wid = sid * 2 + cid
        row0 = wid * rows_per_w

        def idx_load(s, sp):
            return pltpu.make_async_copy(idx_hbm.at[wid, s], ibufs[sp],
                                         isems[sp])

        def gather(h, t, b, ib):
            return pltpu.make_async_copy(table_hbm.at[ib.at[t]], bufs[b],
                                         gsems[b])

        def writes(h, b):
            r = row0 + h // KB
            w0 = (h % KB) * L
            return [
                pltpu.make_async_copy(
                    bufs[b].at[pl.ds(c * L, L), :],
                    out_hbm.at[r, pl.ds(w0, L), pl.ds(c * D, D)],
                    wsems[b])
                for c in range(C)
            ]

        idx_load(0, 0).start()
        idx_load(1, 1).start()

        @pl.loop(0, NS, step=2)
        def _(so):
            for sp in range(2):
                s = so + sp
                ib = ibufs[sp]
                h0 = s * SC_
                idx_load(s, sp).wait()

                @pl.loop(0, SC_)
                def _(t):
                    ib[t, pl.ds(L, L)] = ib[t, pl.ds(L, L)] + VALUES_PER_CHANNEL
                    ib[t, pl.ds(2 * L, L)] = (ib[t, pl.ds(2 * L, L)]
                                              + 2 * VALUES_PER_CHANNEL)

                for t in range(2):
                    h = h0 + t

                    @pl.when(h >= NB)
                    def _():
                        for d_ in writes(h - NB, t):
                            d_.wait()
                    gather(h, t, t, ib).start()

                @pl.loop(0, SC_, step=NB)
                def _(to):
                    for b in range(NB):
                        t = to + b
                        h = h0 + t

                        @pl.when(t + 2 < SC_)
                        def _():
                            b2 = (b + 2) % NB

                            @pl.when(h >= 2)
                            def _():
                                for d_ in writes(h - 2, b2):
                                    d_.wait()
                            gather(h + 2, t + 2, b2, ib).start()

                        gather(h, t, b, ib).wait()
                        for d_ in writes(h, b):
                            d_.start()

                @pl.when(s + 2 < NS)
                def _():
                    idx_load(s + 2, sp).start()

        for t in range(NB):
            h = G - NB + t
            for d_ in writes(h, h % NB):
                d_.wait()

    return k


def kernel(inputs, table):
    idx = inputs.reshape(R, KB, L, C).transpose(0, 1, 3, 2)
    idx = idx.reshape(NW, NS, SC_, K)
    out = _build()(idx, table)
    return out.reshape(inputs.shape[:1] + (224, 224, 768))
